# baseline (speedup 1.0000x reference)
import functools

import jax
import jax.numpy as jnp
from jax import lax
from jax.experimental import pallas as pl
from jax.experimental.pallas import tpu as pltpu
from jax.experimental.pallas import tpu_sc as plsc

N_NODES = 10000
N_EDGES = 320000
NC = 2
NS = 16
NW = NC * NS
K = 125
STEPS = N_EDGES // (NW * K)
ROWS_PER_TILE = N_NODES // NS

_mesh = plsc.VectorSubcoreMesh(core_axis_name="c", subcore_axis_name="s")
_sc_params = pltpu.CompilerParams(use_tc_tiling_on_sc=False)


def _zero_vmem(ref, n_rows, width):
    zero = jnp.zeros((16,), jnp.float32)

    def row(i, _):
        def col(j, _):
            ref[i, pl.ds(j * 16, 16)] = zero
            return 0
        return lax.fori_loop(0, width // 16, col, 0)

    lax.fori_loop(0, n_rows, row, 0)


@functools.partial(
    pl.kernel,
    mesh=_mesh,
    compiler_params=_sc_params,
    out_type=jax.ShapeDtypeStruct((NC, N_NODES, 16), jnp.float32),
    scratch_types=[
        pltpu.VMEM((STEPS, K), jnp.int32),
        pltpu.VMEM((STEPS, K), jnp.int32),
        pltpu.VMEM((K, 16), jnp.float32),
        pltpu.VMEM((K, 16), jnp.float32),
        pltpu.VMEM((ROWS_PER_TILE, 16), jnp.float32),
        pltpu.VMEM_SHARED((N_NODES, 16), jnp.float32),
    ],
)
def _sc_degrees(src2d, dst2d, pat_s, pat_d, out, sidx, didx, ones_s, ones_d,
                zeros, acc):
    c = lax.axis_index("c")
    s = lax.axis_index("s")
    wid = c * NS + s

    _zero_vmem(zeros, ROWS_PER_TILE, 16)
    pltpu.sync_copy(zeros, acc.at[pl.ds(s * ROWS_PER_TILE, ROWS_PER_TILE)])
    pltpu.sync_copy(pat_s, ones_s)
    pltpu.sync_copy(pat_d, ones_d)
    plsc.subcore_barrier()

    base = wid * STEPS
    pltpu.sync_copy(src2d.at[pl.ds(base, STEPS)], sidx)
    pltpu.sync_copy(dst2d.at[pl.ds(base, STEPS)], didx)

    def step(j, _):
        pltpu.sync_copy(ones_s, acc.at[sidx.at[j]], add=True)
        pltpu.sync_copy(ones_d, acc.at[didx.at[j]], add=True)
        return 0

    lax.fori_loop(0, STEPS, step, 0)
    plsc.subcore_barrier()

    row0 = s * ROWS_PER_TILE
    pltpu.sync_copy(acc.at[pl.ds(row0, ROWS_PER_TILE)],
                    out.at[c, pl.ds(row0, ROWS_PER_TILE)])


def _make_sc_aggregate(width):
    @functools.partial(
        pl.kernel,
        mesh=_mesh,
        compiler_params=_sc_params,
        out_type=jax.ShapeDtypeStruct((NC, N_NODES, width), jnp.float32),
        scratch_types=[
            pltpu.VMEM((STEPS // 2, K), jnp.int32),
            pltpu.VMEM((STEPS // 2, K), jnp.int32),
            pltpu.VMEM((K, width), jnp.float32),
            pltpu.VMEM((K, width), jnp.float32),
            pltpu.VMEM_SHARED((N_NODES, width), jnp.float32),
            pltpu.SemaphoreType.DMA,
            pltpu.SemaphoreType.DMA,
        ],
    )
    def agg(table, src2d, dst2d, out, sidx, didx, rows_a, rows_b, acc,
            sem_a, sem_b):
        c = lax.axis_index("c")
        s = lax.axis_index("s")
        wid = c * NS + s
        half = STEPS // 2

        _zero_vmem(rows_a, K, width)
        for r in range(ROWS_PER_TILE // K):
            off = s * ROWS_PER_TILE + r * K
            pltpu.sync_copy(rows_a, acc.at[pl.ds(off, K)])
        plsc.subcore_barrier()

        for h in range(2):
            base = wid * STEPS + h * half
            pltpu.sync_copy(src2d.at[pl.ds(base, half)], sidx)
            pltpu.sync_copy(dst2d.at[pl.ds(base, half)], didx)

            pltpu.async_copy(table.at[sidx.at[0]], rows_a, sem_a)

            def step(i, _):
                ja = 2 * i
                jb = ja + 1
                pltpu.make_async_copy(table.at[sidx.at[ja]], rows_a,
                                      sem_a).wait()
                pltpu.async_copy(table.at[sidx.at[jb]], rows_b, sem_a)
                pltpu.sync_copy(rows_a, acc.at[didx.at[ja]], add=True)
                pltpu.make_async_copy(table.at[sidx.at[jb]], rows_b,
                                      sem_a).wait()
                pltpu.async_copy(table.at[sidx.at[jb + 1]], rows_a, sem_a)
                pltpu.sync_copy(rows_b, acc.at[didx.at[jb]], add=True)
                return 0

            lax.fori_loop(0, half // 2 - 1, step, 0)
            ja = half - 2
            jb = half - 1
            pltpu.make_async_copy(table.at[sidx.at[ja]], rows_a, sem_a).wait()
            pltpu.async_copy(table.at[sidx.at[jb]], rows_b, sem_a)
            pltpu.sync_copy(rows_a, acc.at[didx.at[ja]], add=True)
            pltpu.make_async_copy(table.at[sidx.at[jb]], rows_b, sem_a).wait()
            pltpu.sync_copy(rows_b, acc.at[didx.at[jb]], add=True)
        plsc.subcore_barrier()

        row0 = s * ROWS_PER_TILE
        pltpu.sync_copy(acc.at[pl.ds(row0, ROWS_PER_TILE)],
                        out.at[c, pl.ds(row0, ROWS_PER_TILE)])

    return agg


_sc_agg128 = _make_sc_aggregate(128)
_sc_agg64 = _make_sc_aggregate(64)


_BLK = 1000


def _rsqrt_deg(dref, col):
    d = dref[0, :, col:col + 1] + dref[1, :, col:col + 1]
    return lax.rsqrt(jnp.maximum(d, 1.0))


def _scale_body(x_ref, deg_ref, o_ref):
    o_ref[...] = x_ref[...] * _rsqrt_deg(deg_ref, 0)


def _mid_body(p_ref, deg_ref, w1_ref, b1_ref, w2_ref, o_ref):
    rin = _rsqrt_deg(deg_ref, 8)
    rout = _rsqrt_deg(deg_ref, 0)
    agg = (p_ref[0] + p_ref[1]) * rin
    h = jnp.dot(agg, w1_ref[...], preferred_element_type=jnp.float32)
    h = jnp.maximum(h + b1_ref[...], 0.0) * rout
    o_ref[...] = jnp.dot(h, w2_ref[...], preferred_element_type=jnp.float32)


def _out_body(p_ref, deg_ref, b2_ref, o_ref):
    o_ref[...] = (p_ref[0] + p_ref[1]) * _rsqrt_deg(deg_ref, 8) + b2_ref[...]


def _deg_spec():
    return pl.BlockSpec((2, _BLK, 16), lambda i: (0, i, 0))


def _tc_scale(x, deg):
    return pl.pallas_call(
        _scale_body,
        grid=(N_NODES // _BLK,),
        in_specs=[pl.BlockSpec((_BLK, 128), lambda i: (i, 0)), _deg_spec()],
        out_specs=pl.BlockSpec((_BLK, 128), lambda i: (i, 0)),
        out_shape=jax.ShapeDtypeStruct((N_NODES, 128), jnp.float32),
    )(x, deg)


def _tc_mid(p1, deg, w1, b1, w2):
    return pl.pallas_call(
        _mid_body,
        grid=(N_NODES // _BLK,),
        in_specs=[
            pl.BlockSpec((2, _BLK, 128), lambda i: (0, i, 0)),
            _deg_spec(),
            pl.BlockSpec((128, 128), lambda i: (0, 0)),
            pl.BlockSpec((1, 128), lambda i: (0, 0)),
            pl.BlockSpec((128, 64), lambda i: (0, 0)),
        ],
        out_specs=pl.BlockSpec((_BLK, 64), lambda i: (i, 0)),
        out_shape=jax.ShapeDtypeStruct((N_NODES, 64), jnp.float32),
    )(p1, deg, w1, b1, w2)


def _tc_out(p2, deg, b2):
    return pl.pallas_call(
        _out_body,
        grid=(N_NODES // _BLK,),
        in_specs=[
            pl.BlockSpec((2, _BLK, 64), lambda i: (0, i, 0)),
            _deg_spec(),
            pl.BlockSpec((1, 64), lambda i: (0, 0)),
        ],
        out_specs=pl.BlockSpec((_BLK, 64), lambda i: (i, 0)),
        out_shape=jax.ShapeDtypeStruct((N_NODES, 64), jnp.float32),
    )(p2, deg, b2)


def kernel(x, edge_index, W1, b1, W2, b2):
    ei = edge_index.astype(jnp.int32)
    src2d = ei[0].reshape(NW * STEPS, K)
    dst2d = ei[1].reshape(NW * STEPS, K)

    col = lax.broadcasted_iota(jnp.float32, (K, 16), 1)
    pat_s = (col < 8).astype(jnp.float32)
    pat_d = 1.0 - pat_s

    deg = _sc_degrees(src2d, dst2d, pat_s, pat_d)
    hs = _tc_scale(x, deg)
    p1 = _sc_agg128(hs, src2d, dst2d)
    y2 = _tc_mid(p1, deg, W1, b1.reshape(1, -1), W2)
    p2 = _sc_agg64(y2, src2d, dst2d)
    return _tc_out(p2, deg, b2.reshape(1, -1))


# speedup vs baseline: 1.0231x; 1.0231x over previous
import functools

import jax
import jax.numpy as jnp
from jax import lax
from jax.experimental import pallas as pl
from jax.experimental.pallas import tpu as pltpu
from jax.experimental.pallas import tpu_sc as plsc

N_NODES = 10000
N_EDGES = 320000
NC = 2
NS = 16
NW = NC * NS
K = 125
STEPS = N_EDGES // (NW * K)
ROWS_PER_TILE = N_NODES // NS

_mesh = plsc.VectorSubcoreMesh(core_axis_name="c", subcore_axis_name="s")
_sc_params = pltpu.CompilerParams(use_tc_tiling_on_sc=False)


def _zero_vmem(ref, n_rows, width):
    zero = jnp.zeros((16,), jnp.float32)

    def row(i, _):
        def col(j, _):
            ref[i, pl.ds(j * 16, 16)] = zero
            return 0
        return lax.fori_loop(0, width // 16, col, 0)

    lax.fori_loop(0, n_rows, row, 0)


@functools.partial(
    pl.kernel,
    mesh=_mesh,
    compiler_params=_sc_params,
    out_type=jax.ShapeDtypeStruct((NC, N_NODES, 16), jnp.float32),
    scratch_types=[
        pltpu.VMEM((STEPS, K), jnp.int32),
        pltpu.VMEM((STEPS, K), jnp.int32),
        pltpu.VMEM((K, 16), jnp.float32),
        pltpu.VMEM((K, 16), jnp.float32),
        pltpu.VMEM((ROWS_PER_TILE, 16), jnp.float32),
        pltpu.VMEM_SHARED((N_NODES, 16), jnp.float32),
    ],
)
def _sc_degrees(src2d, dst2d, pat_s, pat_d, out, sidx, didx, ones_s, ones_d,
                zeros, acc):
    c = lax.axis_index("c")
    s = lax.axis_index("s")
    wid = c * NS + s

    _zero_vmem(zeros, ROWS_PER_TILE, 16)
    pltpu.sync_copy(zeros, acc.at[pl.ds(s * ROWS_PER_TILE, ROWS_PER_TILE)])
    pltpu.sync_copy(pat_s, ones_s)
    pltpu.sync_copy(pat_d, ones_d)
    plsc.subcore_barrier()

    base = wid * STEPS
    pltpu.sync_copy(src2d.at[pl.ds(base, STEPS)], sidx)
    pltpu.sync_copy(dst2d.at[pl.ds(base, STEPS)], didx)

    def step(j, _):
        pltpu.sync_copy(ones_s, acc.at[sidx.at[j]], add=True)
        pltpu.sync_copy(ones_d, acc.at[didx.at[j]], add=True)
        return 0

    lax.fori_loop(0, STEPS, step, 0)
    plsc.subcore_barrier()

    row0 = s * ROWS_PER_TILE
    pltpu.sync_copy(acc.at[pl.ds(row0, ROWS_PER_TILE)],
                    out.at[c, pl.ds(row0, ROWS_PER_TILE)])


def _make_sc_aggregate(width, stage_table):
    tbl_scratch = (
        [pltpu.VMEM_SHARED((N_NODES, width), jnp.float32)] if stage_table
        else [])

    @functools.partial(
        pl.kernel,
        mesh=_mesh,
        compiler_params=_sc_params,
        out_type=jax.ShapeDtypeStruct((NC, N_NODES, width), jnp.float32),
        scratch_types=[
            pltpu.VMEM((STEPS // 2, K), jnp.int32),
            pltpu.VMEM((STEPS // 2, K), jnp.int32),
            pltpu.VMEM((K, width), jnp.float32),
            pltpu.VMEM((K, width), jnp.float32),
            pltpu.VMEM_SHARED((N_NODES, width), jnp.float32),
        ] + tbl_scratch + [
            pltpu.SemaphoreType.DMA,
            pltpu.SemaphoreType.DMA,
        ],
    )
    def agg(table, src2d, dst2d, out, sidx, didx, rows_a, rows_b, acc,
            *rest):
        if stage_table:
            tbl, sem_a, sem_b = rest
        else:
            sem_a, sem_b = rest
            tbl = table
        c = lax.axis_index("c")
        s = lax.axis_index("s")
        wid = c * NS + s
        half = STEPS // 2

        if stage_table:
            row0 = s * ROWS_PER_TILE
            pltpu.sync_copy(table.at[pl.ds(row0, ROWS_PER_TILE)],
                            tbl.at[pl.ds(row0, ROWS_PER_TILE)])

        _zero_vmem(rows_a, K, width)
        for r in range(ROWS_PER_TILE // K):
            off = s * ROWS_PER_TILE + r * K
            pltpu.sync_copy(rows_a, acc.at[pl.ds(off, K)])
        plsc.subcore_barrier()

        for h in range(2):
            base = wid * STEPS + h * half
            pltpu.sync_copy(src2d.at[pl.ds(base, half)], sidx)
            pltpu.sync_copy(dst2d.at[pl.ds(base, half)], didx)

            pltpu.async_copy(tbl.at[sidx.at[0]], rows_a, sem_a)

            def step(i, _):
                ja = 2 * i
                jb = ja + 1
                pltpu.make_async_copy(tbl.at[sidx.at[ja]], rows_a,
                                      sem_a).wait()
                pltpu.async_copy(tbl.at[sidx.at[jb]], rows_b, sem_a)
                pltpu.sync_copy(rows_a, acc.at[didx.at[ja]], add=True)
                pltpu.make_async_copy(tbl.at[sidx.at[jb]], rows_b,
                                      sem_a).wait()
                pltpu.async_copy(tbl.at[sidx.at[jb + 1]], rows_a, sem_a)
                pltpu.sync_copy(rows_b, acc.at[didx.at[jb]], add=True)
                return 0

            lax.fori_loop(0, half // 2 - 1, step, 0)
            ja = half - 2
            jb = half - 1
            pltpu.make_async_copy(tbl.at[sidx.at[ja]], rows_a, sem_a).wait()
            pltpu.async_copy(tbl.at[sidx.at[jb]], rows_b, sem_a)
            pltpu.sync_copy(rows_a, acc.at[didx.at[ja]], add=True)
            pltpu.make_async_copy(tbl.at[sidx.at[jb]], rows_b, sem_a).wait()
            pltpu.sync_copy(rows_b, acc.at[didx.at[jb]], add=True)
        plsc.subcore_barrier()

        row0 = s * ROWS_PER_TILE
        pltpu.sync_copy(acc.at[pl.ds(row0, ROWS_PER_TILE)],
                        out.at[c, pl.ds(row0, ROWS_PER_TILE)])

    return agg


_sc_agg128 = _make_sc_aggregate(128, stage_table=False)
_sc_agg64 = _make_sc_aggregate(64, stage_table=True)


_BLK = 1000


def _rsqrt_deg(dref, col):
    d = dref[0, :, col:col + 1] + dref[1, :, col:col + 1]
    return lax.rsqrt(jnp.maximum(d, 1.0))


def _scale_body(x_ref, deg_ref, o_ref):
    o_ref[...] = x_ref[...] * _rsqrt_deg(deg_ref, 0)


def _mid_body(p_ref, deg_ref, w1_ref, b1_ref, w2_ref, o_ref):
    rin = _rsqrt_deg(deg_ref, 8)
    rout = _rsqrt_deg(deg_ref, 0)
    agg = (p_ref[0] + p_ref[1]) * rin
    h = jnp.dot(agg, w1_ref[...], preferred_element_type=jnp.float32)
    h = jnp.maximum(h + b1_ref[...], 0.0) * rout
    o_ref[...] = jnp.dot(h, w2_ref[...], preferred_element_type=jnp.float32)


def _out_body(p_ref, deg_ref, b2_ref, o_ref):
    o_ref[...] = (p_ref[0] + p_ref[1]) * _rsqrt_deg(deg_ref, 8) + b2_ref[...]


def _deg_spec():
    return pl.BlockSpec((2, _BLK, 16), lambda i: (0, i, 0))


def _tc_scale(x, deg):
    return pl.pallas_call(
        _scale_body,
        grid=(N_NODES // _BLK,),
        in_specs=[pl.BlockSpec((_BLK, 128), lambda i: (i, 0)), _deg_spec()],
        out_specs=pl.BlockSpec((_BLK, 128), lambda i: (i, 0)),
        out_shape=jax.ShapeDtypeStruct((N_NODES, 128), jnp.float32),
    )(x, deg)


def _tc_mid(p1, deg, w1, b1, w2):
    return pl.pallas_call(
        _mid_body,
        grid=(N_NODES // _BLK,),
        in_specs=[
            pl.BlockSpec((2, _BLK, 128), lambda i: (0, i, 0)),
            _deg_spec(),
            pl.BlockSpec((128, 128), lambda i: (0, 0)),
            pl.BlockSpec((1, 128), lambda i: (0, 0)),
            pl.BlockSpec((128, 64), lambda i: (0, 0)),
        ],
        out_specs=pl.BlockSpec((_BLK, 64), lambda i: (i, 0)),
        out_shape=jax.ShapeDtypeStruct((N_NODES, 64), jnp.float32),
    )(p1, deg, w1, b1, w2)


def _tc_out(p2, deg, b2):
    return pl.pallas_call(
        _out_body,
        grid=(N_NODES // _BLK,),
        in_specs=[
            pl.BlockSpec((2, _BLK, 64), lambda i: (0, i, 0)),
            _deg_spec(),
            pl.BlockSpec((1, 64), lambda i: (0, 0)),
        ],
        out_specs=pl.BlockSpec((_BLK, 64), lambda i: (i, 0)),
        out_shape=jax.ShapeDtypeStruct((N_NODES, 64), jnp.float32),
    )(p2, deg, b2)


def kernel(x, edge_index, W1, b1, W2, b2):
    ei = edge_index.astype(jnp.int32)
    src2d = ei[0].reshape(NW * STEPS, K)
    dst2d = ei[1].reshape(NW * STEPS, K)

    col = lax.broadcasted_iota(jnp.float32, (K, 16), 1)
    pat_s = (col < 8).astype(jnp.float32)
    pat_d = 1.0 - pat_s

    deg = _sc_degrees(src2d, dst2d, pat_s, pat_d)
    hs = _tc_scale(x, deg)
    p1 = _sc_agg128(hs, src2d, dst2d)
    y2 = _tc_mid(p1, deg, W1, b1.reshape(1, -1), W2)
    p2 = _sc_agg64(y2, src2d, dst2d)
    return _tc_out(p2, deg, b2.reshape(1, -1))
